# speedup vs baseline: 1.0001x; 1.0001x over previous
import jax
import jax.numpy as jnp
from jax.experimental import pallas as pl

N_LAYERS = 12


def _bn(x, g, b):
    mu = jnp.mean(x, axis=0)
    var = jnp.var(x, axis=0)
    return (x - mu) / jnp.sqrt(var + 1e-5) * g + b


def kernel(features, edge_index, edge_weight, eps, W1, b1, W2, b2, bn1_g, bn1_b, bn2_g, bn2_b, pred_W, pred_b):
    src = edge_index[0]
    dst = edge_index[1]
    h = features
    hidden_rep = [h]
    for i in range(N_LAYERS):
        msg = jnp.take(h, src, axis=0) * edge_weight[:, None]
        agg = jnp.zeros((features.shape[0], msg.shape[1]), dtype=h.dtype).at[dst].add(msg)
        x = (1.0 + eps[i]) * h + agg
        z = x @ W1[i] + b1[i]
        z = jax.nn.relu(_bn(z, bn1_g[i], bn1_b[i]))
        z = z @ W2[i] + b2[i]
        h = jax.nn.relu(_bn(z, bn2_g[i], bn2_b[i]))
        hidden_rep.append(h)
    score = jnp.zeros((1, 7), dtype=jnp.float32)
    for i, hr in enumerate(hidden_rep):
        pooled = jnp.sum(hr, axis=0, keepdims=True)
        score = score + (pooled @ pred_W[i] + pred_b[i])
    return score


# speedup vs baseline: 3.4754x; 3.4749x over previous
import functools

import jax
import jax.numpy as jnp
from jax import lax
from jax.experimental import pallas as pl
from jax.experimental.pallas import tpu as pltpu
from jax.experimental.pallas import tpu_sc as plsc

_N = 10000
_D = 128
_NLAYERS = 12
_NW = 32
_C = 128
_CH = 160
_EPAD = _NW * _CH * _C
_RPT = 624
_SB = 32


def _sc_agg_body(h_hbm, src_hbm, dst_hbm, w_hbm, agg_hbm,
                 src_v, dst_v, w_v, rows_v, shared_agg, sem):
    cid = lax.axis_index("c")
    sid = lax.axis_index("s")
    wid = cid * 16 + sid

    zv = jnp.zeros((16,), jnp.float32)

    def zrow(e, carry):
        for j in range(8):
            rows_v[e, pl.ds(j * 16, 16)] = zv
        return carry

    lax.fori_loop(0, _C, zrow, 0)
    for k in range(6):
        pltpu.sync_copy(rows_v.at[pl.ds(0, 104)],
                        shared_agg.at[pl.ds(sid * _RPT + k * 104, 104)])

    @pl.when(sid == 15)
    def _zero_tail():
        pltpu.sync_copy(rows_v.at[pl.ds(0, 16)],
                        shared_agg.at[pl.ds(16 * _RPT, 16)])

    plsc.subcore_barrier()

    def superchunk(s, carry):
        pltpu.sync_copy(src_hbm.at[wid, pl.ds(s * _SB, _SB)], src_v)
        pltpu.sync_copy(dst_hbm.at[wid, pl.ds(s * _SB, _SB)], dst_v)
        pltpu.sync_copy(w_hbm.at[wid, pl.ds(s * _SB, _SB)], w_v)

        def chunk(c, ccarry):
            pltpu.async_copy(h_hbm.at[src_v.at[c]], rows_v, sem).wait()

            def grp(g, gcarry):
                wv = w_v[c, pl.ds(g * 16, 16)]
                for l in range(16):
                    vv = jnp.full((16,), wv[l], jnp.float32)
                    e = g * 16 + l
                    for j in range(8):
                        sl = pl.ds(j * 16, 16)
                        rows_v[e, sl] = rows_v[e, sl] * vv
                return gcarry

            lax.fori_loop(0, _C // 16, grp, 0)

            pltpu.sync_copy(rows_v, shared_agg.at[dst_v.at[c]], add=True)
            return ccarry

        lax.fori_loop(0, _SB, chunk, 0)
        return carry

    lax.fori_loop(0, _CH // _SB, superchunk, 0)
    plsc.subcore_barrier()

    pltpu.sync_copy(shared_agg.at[pl.ds(sid * _RPT, _RPT)],
                    agg_hbm.at[cid, pl.ds(sid * _RPT, _RPT)])

    @pl.when(sid == 15)
    def _write_tail():
        pltpu.sync_copy(shared_agg.at[pl.ds(16 * _RPT, 16)],
                        agg_hbm.at[cid, pl.ds(16 * _RPT, 16)])


_sc_agg = functools.partial(
    pl.kernel,
    out_type=jax.ShapeDtypeStruct((2, _N, _D), jnp.float32),
    mesh=plsc.VectorSubcoreMesh(core_axis_name="c", subcore_axis_name="s",
                                num_cores=2, num_subcores=16),
    scratch_types=[
        pltpu.VMEM((_SB, _C), jnp.int32),
        pltpu.VMEM((_SB, _C), jnp.int32),
        pltpu.VMEM((_SB, _C), jnp.float32),
        pltpu.VMEM((_C, _D), jnp.float32),
        pltpu.VMEM_SHARED((_N, _D), jnp.float32),
        pltpu.SemaphoreType.DMA,
    ],
)(_sc_agg_body)


def _mlp_body(eps_ref, h_ref, agg_ref, w1_ref, b1_ref, w2_ref, b2_ref,
              g1_ref, be1_ref, g2_ref, be2_ref, hout_ref, pin_ref, pout_ref):
    h = h_ref[...]
    x = (1.0 + eps_ref[0, 0]) * h + agg_ref[0] + agg_ref[1]
    z = jnp.dot(x, w1_ref[...], preferred_element_type=jnp.float32) + b1_ref[...]
    mu = jnp.mean(z, axis=0, keepdims=True)
    var = jnp.mean((z - mu) * (z - mu), axis=0, keepdims=True)
    z = (z - mu) / jnp.sqrt(var + 1e-5) * g1_ref[...] + be1_ref[...]
    z = jnp.maximum(z, 0.0)
    z2 = jnp.dot(z, w2_ref[...], preferred_element_type=jnp.float32) + b2_ref[...]
    mu2 = jnp.mean(z2, axis=0, keepdims=True)
    var2 = jnp.mean((z2 - mu2) * (z2 - mu2), axis=0, keepdims=True)
    ho = (z2 - mu2) / jnp.sqrt(var2 + 1e-5) * g2_ref[...] + be2_ref[...]
    ho = jnp.maximum(ho, 0.0)
    hout_ref[...] = ho
    pin_ref[...] = jnp.sum(h, axis=0, keepdims=True)
    pout_ref[...] = jnp.sum(ho, axis=0, keepdims=True)


_mlp = pl.pallas_call(
    _mlp_body,
    out_shape=[
        jax.ShapeDtypeStruct((_N, _D), jnp.float32),
        jax.ShapeDtypeStruct((1, _D), jnp.float32),
        jax.ShapeDtypeStruct((1, _D), jnp.float32),
    ],
    in_specs=[pl.BlockSpec(memory_space=pltpu.SMEM)]
    + [pl.BlockSpec(memory_space=pltpu.VMEM)] * 10,
    out_specs=[pl.BlockSpec(memory_space=pltpu.VMEM)] * 3,
)


def kernel(features, edge_index, edge_weight, eps, W1, b1, W2, b2,
           bn1_g, bn1_b, bn2_g, bn2_b, pred_W, pred_b):
    src = jnp.pad(edge_index[0].astype(jnp.int32), (0, _EPAD - edge_weight.shape[0]))
    dst = jnp.pad(edge_index[1].astype(jnp.int32), (0, _EPAD - edge_weight.shape[0]))
    w = jnp.pad(edge_weight, (0, _EPAD - edge_weight.shape[0]))
    src = src.reshape(_NW, _CH, _C)
    dst = dst.reshape(_NW, _CH, _C)
    w = w.reshape(_NW, _CH, _C)

    h = features
    pooled = []
    for i in range(_NLAYERS):
        agg = _sc_agg(h, src, dst, w)
        h, p_in, p_out = _mlp(
            eps[i].reshape(1, 1), h, agg,
            W1[i], b1[i][None], W2[i], b2[i][None],
            bn1_g[i][None], bn1_b[i][None], bn2_g[i][None], bn2_b[i][None],
        )
        if i == 0:
            pooled.append(p_in)
        pooled.append(p_out)

    P = jnp.concatenate(pooled, axis=0)
    score = jnp.einsum("ld,ldo->o", P, pred_W) + jnp.sum(pred_b, axis=0)
    return score[None, :]


# speedup vs baseline: 4.0830x; 1.1748x over previous
import functools

import jax
import jax.numpy as jnp
from jax import lax
from jax.experimental import pallas as pl
from jax.experimental.pallas import tpu as pltpu
from jax.experimental.pallas import tpu_sc as plsc

_N = 10000
_D = 128
_NLAYERS = 12
_NW = 32
_C = 128
_CH = 160
_EPAD = _NW * _CH * _C
_RPT = 624
_SB = 32


def _sc_agg_body(h_hbm, src_hbm, dst_hbm, w_hbm, agg_hbm,
                 src_v, dst_v, w_v, rows0, rows1, shared_agg,
                 g0, g1, s0, s1):
    cid = lax.axis_index("c")
    sid = lax.axis_index("s")
    wid = cid * 16 + sid

    zv = jnp.zeros((16,), jnp.float32)

    def zrow(e, carry):
        for j in range(8):
            rows0[e, pl.ds(j * 16, 16)] = zv
        return carry

    lax.fori_loop(0, _C, zrow, 0)
    for k in range(6):
        pltpu.sync_copy(rows0.at[pl.ds(0, 104)],
                        shared_agg.at[pl.ds(sid * _RPT + k * 104, 104)])

    @pl.when(sid == 15)
    def _zero_tail():
        pltpu.sync_copy(rows0.at[pl.ds(0, 16)],
                        shared_agg.at[pl.ds(16 * _RPT, 16)])

    plsc.subcore_barrier()

    def scale(rows, c):
        def grp(g, gcarry):
            wv = w_v[c, pl.ds(g * 16, 16)]
            for l in range(16):
                vv = jnp.full((16,), wv[l], jnp.float32)
                e = g * 16 + l
                for j in range(8):
                    sl = pl.ds(j * 16, 16)
                    rows[e, sl] = rows[e, sl] * vv
            return gcarry

        lax.fori_loop(0, _C // 16, grp, 0)

    def gather(rows, c, sem):
        pltpu.async_copy(h_hbm.at[src_v.at[c]], rows, sem)

    def gather_wait(rows, sem):
        pltpu.make_async_copy(h_hbm.at[src_v.at[0]], rows, sem).wait()

    def scatter(rows, c, sem):
        pltpu.async_copy(rows, shared_agg.at[dst_v.at[c]], sem, add=True)

    def scatter_wait(rows, sem):
        pltpu.make_async_copy(rows, shared_agg.at[dst_v.at[0]], sem).wait()

    def superchunk(s, carry):
        pltpu.sync_copy(src_hbm.at[wid, pl.ds(s * _SB, _SB)], src_v)
        pltpu.sync_copy(dst_hbm.at[wid, pl.ds(s * _SB, _SB)], dst_v)
        pltpu.sync_copy(w_hbm.at[wid, pl.ds(s * _SB, _SB)], w_v)
        gather(rows0, 0, g0)

        def pair(k, pcarry):
            c0 = 2 * k
            gather_wait(rows0, g0)

            @pl.when(k > 0)
            def _drain_s1():
                scatter_wait(rows1, s1)

            gather(rows1, c0 + 1, g1)
            scale(rows0, c0)
            scatter(rows0, c0, s0)
            gather_wait(rows1, g1)
            scatter_wait(rows0, s0)

            @pl.when(k < _SB // 2 - 1)
            def _prefetch():
                gather(rows0, c0 + 2, g0)

            scale(rows1, c0 + 1)
            scatter(rows1, c0 + 1, s1)
            return pcarry

        lax.fori_loop(0, _SB // 2, pair, 0)
        scatter_wait(rows1, s1)
        return carry

    lax.fori_loop(0, _CH // _SB, superchunk, 0)
    plsc.subcore_barrier()

    pltpu.sync_copy(shared_agg.at[pl.ds(sid * _RPT, _RPT)],
                    agg_hbm.at[cid, pl.ds(sid * _RPT, _RPT)])

    @pl.when(sid == 15)
    def _write_tail():
        pltpu.sync_copy(shared_agg.at[pl.ds(16 * _RPT, 16)],
                        agg_hbm.at[cid, pl.ds(16 * _RPT, 16)])


_sc_agg = functools.partial(
    pl.kernel,
    out_type=jax.ShapeDtypeStruct((2, _N, _D), jnp.float32),
    mesh=plsc.VectorSubcoreMesh(core_axis_name="c", subcore_axis_name="s",
                                num_cores=2, num_subcores=16),
    scratch_types=[
        pltpu.VMEM((_SB, _C), jnp.int32),
        pltpu.VMEM((_SB, _C), jnp.int32),
        pltpu.VMEM((_SB, _C), jnp.float32),
        pltpu.VMEM((_C, _D), jnp.float32),
        pltpu.VMEM((_C, _D), jnp.float32),
        pltpu.VMEM_SHARED((_N, _D), jnp.float32),
        pltpu.SemaphoreType.DMA,
        pltpu.SemaphoreType.DMA,
        pltpu.SemaphoreType.DMA,
        pltpu.SemaphoreType.DMA,
    ],
)(_sc_agg_body)


def _mlp_body(eps_ref, h_ref, agg_ref, w1_ref, b1_ref, w2_ref, b2_ref,
              g1_ref, be1_ref, g2_ref, be2_ref, hout_ref, pin_ref, pout_ref):
    h = h_ref[...]
    x = (1.0 + eps_ref[0, 0]) * h + agg_ref[0] + agg_ref[1]
    z = jnp.dot(x, w1_ref[...], preferred_element_type=jnp.float32) + b1_ref[...]
    mu = jnp.mean(z, axis=0, keepdims=True)
    var = jnp.mean((z - mu) * (z - mu), axis=0, keepdims=True)
    z = (z - mu) / jnp.sqrt(var + 1e-5) * g1_ref[...] + be1_ref[...]
    z = jnp.maximum(z, 0.0)
    z2 = jnp.dot(z, w2_ref[...], preferred_element_type=jnp.float32) + b2_ref[...]
    mu2 = jnp.mean(z2, axis=0, keepdims=True)
    var2 = jnp.mean((z2 - mu2) * (z2 - mu2), axis=0, keepdims=True)
    ho = (z2 - mu2) / jnp.sqrt(var2 + 1e-5) * g2_ref[...] + be2_ref[...]
    ho = jnp.maximum(ho, 0.0)
    hout_ref[...] = ho
    pin_ref[...] = jnp.sum(h, axis=0, keepdims=True)
    pout_ref[...] = jnp.sum(ho, axis=0, keepdims=True)


_mlp = pl.pallas_call(
    _mlp_body,
    out_shape=[
        jax.ShapeDtypeStruct((_N, _D), jnp.float32),
        jax.ShapeDtypeStruct((1, _D), jnp.float32),
        jax.ShapeDtypeStruct((1, _D), jnp.float32),
    ],
    in_specs=[pl.BlockSpec(memory_space=pltpu.SMEM)]
    + [pl.BlockSpec(memory_space=pltpu.VMEM)] * 10,
    out_specs=[pl.BlockSpec(memory_space=pltpu.VMEM)] * 3,
)


def kernel(features, edge_index, edge_weight, eps, W1, b1, W2, b2,
           bn1_g, bn1_b, bn2_g, bn2_b, pred_W, pred_b):
    src = jnp.pad(edge_index[0].astype(jnp.int32), (0, _EPAD - edge_weight.shape[0]))
    dst = jnp.pad(edge_index[1].astype(jnp.int32), (0, _EPAD - edge_weight.shape[0]))
    w = jnp.pad(edge_weight, (0, _EPAD - edge_weight.shape[0]))
    src = src.reshape(_NW, _CH, _C)
    dst = dst.reshape(_NW, _CH, _C)
    w = w.reshape(_NW, _CH, _C)

    h = features
    pooled = []
    for i in range(_NLAYERS):
        agg = _sc_agg(h, src, dst, w)
        h, p_in, p_out = _mlp(
            eps[i].reshape(1, 1), h, agg,
            W1[i], b1[i][None], W2[i], b2[i][None],
            bn1_g[i][None], bn1_b[i][None], bn2_g[i][None], bn2_b[i][None],
        )
        if i == 0:
            pooled.append(p_in)
        pooled.append(p_out)

    P = jnp.concatenate(pooled, axis=0)
    score = jnp.einsum("ld,ldo->o", P, pred_W) + jnp.sum(pred_b, axis=0)
    return score[None, :]


# speedup vs baseline: 6.1148x; 1.4976x over previous
import functools

import jax
import jax.numpy as jnp
from jax import lax
from jax.experimental import pallas as pl
from jax.experimental.pallas import tpu as pltpu
from jax.experimental.pallas import tpu_sc as plsc

_N = 10000
_D = 128
_H = 64
_NP = _N // 2
_NLAYERS = 12
_NT = 16
_C = 128
_CH = 320
_EPAD = _NT * _CH * _C
_SB = 32
_RPT = 312


def _sc_agg_body(h_hbm, src_hbm, dst_hbm, par_hbm, w_hbm, agg_hbm,
                 src_v, dst_v, par_v, w_v, rows0, rows1,
                 shared_h, shared_agg, g0, g1, s0, s1):
    cid = lax.axis_index("c")
    sid = lax.axis_index("s")

    for k, nr in ((0, 128), (1, 128), (2, 56)):
        pltpu.sync_copy(h_hbm.at[cid, pl.ds(sid * _RPT + k * 128, nr)],
                        rows1.at[pl.ds(0, nr)])
        pltpu.sync_copy(rows1.at[pl.ds(0, nr)],
                        shared_h.at[pl.ds(sid * _RPT + k * 128, nr)])
    zv = jnp.zeros((16,), jnp.float32)

    def zrow(e, carry):
        for j in range(8):
            rows0[e, pl.ds(j * 16, 16)] = zv
        return carry

    lax.fori_loop(0, _C, zrow, 0)
    for k in range(3):
        pltpu.sync_copy(rows0.at[pl.ds(0, 104)],
                        shared_agg.at[pl.ds(sid * _RPT + k * 104, 104)])

    @pl.when(sid == 15)
    def _stage_tail():
        pltpu.sync_copy(h_hbm.at[cid, pl.ds(16 * _RPT, 8)],
                        rows1.at[pl.ds(0, 8)])
        pltpu.sync_copy(rows1.at[pl.ds(0, 8)],
                        shared_h.at[pl.ds(16 * _RPT, 8)])
        pltpu.sync_copy(rows0.at[pl.ds(0, 8)],
                        shared_agg.at[pl.ds(16 * _RPT, 8)])

    plsc.subcore_barrier()

    def scale(rows, c):
        def grp(g, gcarry):
            wv = w_v[c, pl.ds(g * 16, 16)]
            pv = par_v[c, pl.ds(g * 16, 16)]
            for l in range(16):
                vv = jnp.full((16,), wv[l], jnp.float32)
                par = pv[l]
                ps = 64 * (par & 1)
                pd = 64 * (par >> 1)
                e = g * 16 + l
                vals = [rows[e, pl.ds(ps + j * 16, 16)] * vv for j in range(4)]
                for j in range(4):
                    rows[e, pl.ds(64 - pd + j * 16, 16)] = zv
                for j in range(4):
                    rows[e, pl.ds(pd + j * 16, 16)] = vals[j]
            return gcarry

        lax.fori_loop(0, _C // 16, grp, 0)

    def gather(rows, c, sem):
        pltpu.async_copy(shared_h.at[src_v.at[c]], rows, sem)

    def gather_wait(rows, sem):
        pltpu.make_async_copy(shared_h.at[src_v.at[0]], rows, sem).wait()

    def scatter(rows, c, sem):
        pltpu.async_copy(rows, shared_agg.at[dst_v.at[c]], sem, add=True)

    def scatter_wait(rows, sem):
        pltpu.make_async_copy(rows, shared_agg.at[dst_v.at[0]], sem).wait()

    def superchunk(s, carry):
        pltpu.sync_copy(src_hbm.at[sid, pl.ds(s * _SB, _SB)], src_v)
        pltpu.sync_copy(dst_hbm.at[sid, pl.ds(s * _SB, _SB)], dst_v)
        pltpu.sync_copy(par_hbm.at[sid, pl.ds(s * _SB, _SB)], par_v)
        pltpu.sync_copy(w_hbm.at[sid, pl.ds(s * _SB, _SB)], w_v)
        gather(rows0, 0, g0)

        def pair(k, pcarry):
            c0 = 2 * k
            gather_wait(rows0, g0)

            @pl.when(k > 0)
            def _drain_s1():
                scatter_wait(rows1, s1)

            gather(rows1, c0 + 1, g1)
            scale(rows0, c0)
            scatter(rows0, c0, s0)
            gather_wait(rows1, g1)
            scatter_wait(rows0, s0)

            @pl.when(k < _SB // 2 - 1)
            def _prefetch():
                gather(rows0, c0 + 2, g0)

            scale(rows1, c0 + 1)
            scatter(rows1, c0 + 1, s1)
            return pcarry

        lax.fori_loop(0, _SB // 2, pair, 0)
        scatter_wait(rows1, s1)
        return carry

    lax.fori_loop(0, _CH // _SB, superchunk, 0)
    plsc.subcore_barrier()

    pltpu.sync_copy(shared_agg.at[pl.ds(sid * _RPT, _RPT)],
                    agg_hbm.at[cid, pl.ds(sid * _RPT, _RPT)])

    @pl.when(sid == 15)
    def _write_tail():
        pltpu.sync_copy(shared_agg.at[pl.ds(16 * _RPT, 8)],
                        agg_hbm.at[cid, pl.ds(16 * _RPT, 8)])


_sc_agg = functools.partial(
    pl.kernel,
    out_type=jax.ShapeDtypeStruct((2, _NP, _D), jnp.float32),
    mesh=plsc.VectorSubcoreMesh(core_axis_name="c", subcore_axis_name="s",
                                num_cores=2, num_subcores=16),
    scratch_types=[
        pltpu.VMEM((_SB, _C), jnp.int32),
        pltpu.VMEM((_SB, _C), jnp.int32),
        pltpu.VMEM((_SB, _C), jnp.int32),
        pltpu.VMEM((_SB, _C), jnp.float32),
        pltpu.VMEM((_C, _D), jnp.float32),
        pltpu.VMEM((_C, _D), jnp.float32),
        pltpu.VMEM_SHARED((_NP, _D), jnp.float32),
        pltpu.VMEM_SHARED((_NP, _D), jnp.float32),
        pltpu.SemaphoreType.DMA,
        pltpu.SemaphoreType.DMA,
        pltpu.SemaphoreType.DMA,
        pltpu.SemaphoreType.DMA,
    ],
)(_sc_agg_body)


def _mlp_body(eps_ref, h_ref, agg_ref, w1_ref, b1_ref, w2_ref, b2_ref,
              g1_ref, be1_ref, g2_ref, be2_ref, hout_ref, pin_ref, pout_ref):
    h = h_ref[...]
    agg = agg_ref[...]
    x = (1.0 + eps_ref[0, 0]) * h + agg
    z = jnp.dot(x, w1_ref[...], preferred_element_type=jnp.float32) + b1_ref[...]
    mu = jnp.mean(z, axis=0, keepdims=True)
    var = jnp.mean((z - mu) * (z - mu), axis=0, keepdims=True)
    z = (z - mu) / jnp.sqrt(var + 1e-5) * g1_ref[...] + be1_ref[...]
    z = jnp.maximum(z, 0.0)
    z2 = jnp.dot(z, w2_ref[...], preferred_element_type=jnp.float32) + b2_ref[...]
    mu2 = jnp.mean(z2, axis=0, keepdims=True)
    var2 = jnp.mean((z2 - mu2) * (z2 - mu2), axis=0, keepdims=True)
    ho = (z2 - mu2) / jnp.sqrt(var2 + 1e-5) * g2_ref[...] + be2_ref[...]
    ho = jnp.maximum(ho, 0.0)
    hout_ref[...] = ho
    pin_ref[...] = jnp.sum(h, axis=0, keepdims=True)
    pout_ref[...] = jnp.sum(ho, axis=0, keepdims=True)


_mlp = pl.pallas_call(
    _mlp_body,
    out_shape=[
        jax.ShapeDtypeStruct((_N, _D), jnp.float32),
        jax.ShapeDtypeStruct((1, _D), jnp.float32),
        jax.ShapeDtypeStruct((1, _D), jnp.float32),
    ],
    in_specs=[pl.BlockSpec(memory_space=pltpu.SMEM)]
    + [pl.BlockSpec(memory_space=pltpu.VMEM)] * 10,
    out_specs=[pl.BlockSpec(memory_space=pltpu.VMEM)] * 3,
)


def _split_pairs(h):
    return jnp.stack([h[:, :_H].reshape(_NP, _D), h[:, _H:].reshape(_NP, _D)])


def _merge_pairs(aggp):
    return jnp.concatenate(
        [aggp[0].reshape(_N, _H), aggp[1].reshape(_N, _H)], axis=1)


def kernel(features, edge_index, edge_weight, eps, W1, b1, W2, b2,
           bn1_g, bn1_b, bn2_g, bn2_b, pred_W, pred_b):
    npad = _EPAD - edge_weight.shape[0]
    src = jnp.pad(edge_index[0].astype(jnp.int32), (0, npad))
    dst = jnp.pad(edge_index[1].astype(jnp.int32), (0, npad))
    w = jnp.pad(edge_weight, (0, npad)).reshape(_NT, _CH, _C)
    par = ((src & 1) | ((dst & 1) << 1)).reshape(_NT, _CH, _C)
    srcrow = (src >> 1).reshape(_NT, _CH, _C)
    dstrow = (dst >> 1).reshape(_NT, _CH, _C)

    h = features
    pooled = []
    for i in range(_NLAYERS):
        aggp = _sc_agg(_split_pairs(h), srcrow, dstrow, par, w)
        h, p_in, p_out = _mlp(
            eps[i].reshape(1, 1), h, _merge_pairs(aggp),
            W1[i], b1[i][None], W2[i], b2[i][None],
            bn1_g[i][None], bn1_b[i][None], bn2_g[i][None], bn2_b[i][None],
        )
        if i == 0:
            pooled.append(p_in)
        pooled.append(p_out)

    P = jnp.concatenate(pooled, axis=0)
    score = jnp.einsum("ld,ldo->o", P, pred_W) + jnp.sum(pred_b, axis=0)
    return score[None, :]


# speedup vs baseline: 6.2558x; 1.0231x over previous
import functools

import jax
import jax.numpy as jnp
from jax import lax
from jax.experimental import pallas as pl
from jax.experimental.pallas import tpu as pltpu
from jax.experimental.pallas import tpu_sc as plsc

_N = 10000
_D = 128
_H = 64
_NP = _N // 2
_NLAYERS = 12
_NT = 16
_C = 128
_CH = 320
_EPAD = _NT * _CH * _C
_SB = 32
_RPT = 312


def _sc_agg_body(h_hbm, src_hbm, dst_hbm, par_hbm, w_hbm, agg_hbm,
                 src_v, dst_v, par_v, w_v, rows0, rows1,
                 shared_h, shared_agg, g0, g1, s0, s1):
    cid = lax.axis_index("c")
    sid = lax.axis_index("s")

    for k, nr in ((0, 128), (1, 128), (2, 56)):
        pltpu.sync_copy(h_hbm.at[cid, pl.ds(sid * _RPT + k * 128, nr)],
                        rows1.at[pl.ds(0, nr)])
        pltpu.sync_copy(rows1.at[pl.ds(0, nr)],
                        shared_h.at[pl.ds(sid * _RPT + k * 128, nr)])
    zv = jnp.zeros((16,), jnp.float32)

    def zrow(e, carry):
        for j in range(8):
            rows0[e, pl.ds(j * 16, 16)] = zv
        return carry

    lax.fori_loop(0, _C, zrow, 0)
    for k in range(3):
        pltpu.sync_copy(rows0.at[pl.ds(0, 104)],
                        shared_agg.at[pl.ds(sid * _RPT + k * 104, 104)])

    @pl.when(sid == 15)
    def _stage_tail():
        pltpu.sync_copy(h_hbm.at[cid, pl.ds(16 * _RPT, 8)],
                        rows1.at[pl.ds(0, 8)])
        pltpu.sync_copy(rows1.at[pl.ds(0, 8)],
                        shared_h.at[pl.ds(16 * _RPT, 8)])
        pltpu.sync_copy(rows0.at[pl.ds(0, 8)],
                        shared_agg.at[pl.ds(16 * _RPT, 8)])

    plsc.subcore_barrier()

    def scale(rows, c):
        def grp(g, gcarry):
            wv = w_v[c, pl.ds(g * 16, 16)]
            pv = par_v[c, pl.ds(g * 16, 16)]
            for l in range(16):
                vv = jnp.full((16,), wv[l], jnp.float32)
                par = pv[l]
                ps = 64 * (par & 1)
                pd = 64 * (par >> 1)
                e = g * 16 + l
                vals = [rows[e, pl.ds(ps + j * 16, 16)] * vv for j in range(4)]
                for j in range(4):
                    rows[e, pl.ds(64 - pd + j * 16, 16)] = zv
                for j in range(4):
                    rows[e, pl.ds(pd + j * 16, 16)] = vals[j]
            return gcarry

        lax.fori_loop(0, _C // 16, grp, 0)

    def gather(rows, c, sem):
        pltpu.async_copy(shared_h.at[src_v.at[c]], rows, sem)

    def gather_wait(rows, sem):
        pltpu.make_async_copy(shared_h.at[src_v.at[0]], rows, sem).wait()

    def scatter(rows, c, sem):
        pltpu.async_copy(rows, shared_agg.at[dst_v.at[c]], sem, add=True)

    def scatter_wait(rows, sem):
        pltpu.make_async_copy(rows, shared_agg.at[dst_v.at[0]], sem).wait()

    def superchunk(s, carry):
        pltpu.sync_copy(src_hbm.at[sid, pl.ds(s * _SB, _SB)], src_v)
        pltpu.sync_copy(dst_hbm.at[sid, pl.ds(s * _SB, _SB)], dst_v)
        pltpu.sync_copy(par_hbm.at[sid, pl.ds(s * _SB, _SB)], par_v)
        pltpu.sync_copy(w_hbm.at[sid, pl.ds(s * _SB, _SB)], w_v)
        gather(rows0, 0, g0)

        def pair(k, pcarry):
            c0 = 2 * k
            gather_wait(rows0, g0)

            @pl.when(k > 0)
            def _drain_s1():
                scatter_wait(rows1, s1)

            gather(rows1, c0 + 1, g1)
            scale(rows0, c0)
            scatter(rows0, c0, s0)
            gather_wait(rows1, g1)
            scale(rows1, c0 + 1)
            scatter_wait(rows0, s0)

            @pl.when(k < _SB // 2 - 1)
            def _prefetch():
                gather(rows0, c0 + 2, g0)

            scatter(rows1, c0 + 1, s1)
            return pcarry

        lax.fori_loop(0, _SB // 2, pair, 0)
        scatter_wait(rows1, s1)
        return carry

    lax.fori_loop(0, _CH // _SB, superchunk, 0)
    plsc.subcore_barrier()

    pltpu.sync_copy(shared_agg.at[pl.ds(sid * _RPT, _RPT)],
                    agg_hbm.at[cid, pl.ds(sid * _RPT, _RPT)])

    @pl.when(sid == 15)
    def _write_tail():
        pltpu.sync_copy(shared_agg.at[pl.ds(16 * _RPT, 8)],
                        agg_hbm.at[cid, pl.ds(16 * _RPT, 8)])


_sc_agg = functools.partial(
    pl.kernel,
    out_type=jax.ShapeDtypeStruct((2, _NP, _D), jnp.float32),
    mesh=plsc.VectorSubcoreMesh(core_axis_name="c", subcore_axis_name="s",
                                num_cores=2, num_subcores=16),
    scratch_types=[
        pltpu.VMEM((_SB, _C), jnp.int32),
        pltpu.VMEM((_SB, _C), jnp.int32),
        pltpu.VMEM((_SB, _C), jnp.int32),
        pltpu.VMEM((_SB, _C), jnp.float32),
        pltpu.VMEM((_C, _D), jnp.float32),
        pltpu.VMEM((_C, _D), jnp.float32),
        pltpu.VMEM_SHARED((_NP, _D), jnp.float32),
        pltpu.VMEM_SHARED((_NP, _D), jnp.float32),
        pltpu.SemaphoreType.DMA,
        pltpu.SemaphoreType.DMA,
        pltpu.SemaphoreType.DMA,
        pltpu.SemaphoreType.DMA,
    ],
)(_sc_agg_body)


def _mlp_body(eps_ref, h_ref, agg_ref, w1_ref, b1_ref, w2_ref, b2_ref,
              g1_ref, be1_ref, g2_ref, be2_ref, hout_ref, pin_ref, pout_ref):
    h = h_ref[...]
    agg = agg_ref[...]
    x = (1.0 + eps_ref[0, 0]) * h + agg
    z = jnp.dot(x, w1_ref[...], preferred_element_type=jnp.float32) + b1_ref[...]
    mu = jnp.mean(z, axis=0, keepdims=True)
    var = jnp.mean((z - mu) * (z - mu), axis=0, keepdims=True)
    z = (z - mu) / jnp.sqrt(var + 1e-5) * g1_ref[...] + be1_ref[...]
    z = jnp.maximum(z, 0.0)
    z2 = jnp.dot(z, w2_ref[...], preferred_element_type=jnp.float32) + b2_ref[...]
    mu2 = jnp.mean(z2, axis=0, keepdims=True)
    var2 = jnp.mean((z2 - mu2) * (z2 - mu2), axis=0, keepdims=True)
    ho = (z2 - mu2) / jnp.sqrt(var2 + 1e-5) * g2_ref[...] + be2_ref[...]
    ho = jnp.maximum(ho, 0.0)
    hout_ref[...] = ho
    pin_ref[...] = jnp.sum(h, axis=0, keepdims=True)
    pout_ref[...] = jnp.sum(ho, axis=0, keepdims=True)


_mlp = pl.pallas_call(
    _mlp_body,
    out_shape=[
        jax.ShapeDtypeStruct((_N, _D), jnp.float32),
        jax.ShapeDtypeStruct((1, _D), jnp.float32),
        jax.ShapeDtypeStruct((1, _D), jnp.float32),
    ],
    in_specs=[pl.BlockSpec(memory_space=pltpu.SMEM)]
    + [pl.BlockSpec(memory_space=pltpu.VMEM)] * 10,
    out_specs=[pl.BlockSpec(memory_space=pltpu.VMEM)] * 3,
)


def _split_pairs(h):
    return jnp.stack([h[:, :_H].reshape(_NP, _D), h[:, _H:].reshape(_NP, _D)])


def _merge_pairs(aggp):
    return jnp.concatenate(
        [aggp[0].reshape(_N, _H), aggp[1].reshape(_N, _H)], axis=1)


def kernel(features, edge_index, edge_weight, eps, W1, b1, W2, b2,
           bn1_g, bn1_b, bn2_g, bn2_b, pred_W, pred_b):
    npad = _EPAD - edge_weight.shape[0]
    src = jnp.pad(edge_index[0].astype(jnp.int32), (0, npad))
    dst = jnp.pad(edge_index[1].astype(jnp.int32), (0, npad))
    w = jnp.pad(edge_weight, (0, npad)).reshape(_NT, _CH, _C)
    par = ((src & 1) | ((dst & 1) << 1)).reshape(_NT, _CH, _C)
    srcrow = (src >> 1).reshape(_NT, _CH, _C)
    dstrow = (dst >> 1).reshape(_NT, _CH, _C)

    h = features
    pooled = []
    for i in range(_NLAYERS):
        aggp = _sc_agg(_split_pairs(h), srcrow, dstrow, par, w)
        h, p_in, p_out = _mlp(
            eps[i].reshape(1, 1), h, _merge_pairs(aggp),
            W1[i], b1[i][None], W2[i], b2[i][None],
            bn1_g[i][None], bn1_b[i][None], bn2_g[i][None], bn2_b[i][None],
        )
        if i == 0:
            pooled.append(p_in)
        pooled.append(p_out)

    P = jnp.concatenate(pooled, axis=0)
    score = jnp.einsum("ld,ldo->o", P, pred_W) + jnp.sum(pred_b, axis=0)
    return score[None, :]
